# baseline (speedup 1.0000x reference)
import functools

import jax
import jax.numpy as jnp
from jax import lax
from jax.experimental import pallas as pl
from jax.experimental.pallas import tpu as pltpu
from jax.experimental.pallas import tpu_sc as plsc

_VOCAB = 100000
_EMB = 32
_B = 1024
_L = 200

_NC = 2
_NS = 16
_NW = _NC * _NS
_CH = 100
_LANES = 16
_NHALF = 2
_BH = _B // _NHALF
_BPW = _BH // _NW
_CPW = _BPW * _L // _CH

_mesh = plsc.VectorSubcoreMesh(core_axis_name="c", subcore_axis_name="s")


def _make_pool(half):
    base_chunk = half * (_BH * _L // _CH)

    @functools.partial(
        pl.kernel,
        mesh=_mesh,
        out_type=jax.ShapeDtypeStruct((_BH, _EMB), jnp.float32),
        scratch_types=[
            pltpu.VMEM((_CPW, _CH), jnp.int32),
            pltpu.VMEM((_CH, _EMB), jnp.float32),
            pltpu.VMEM((_CH, _EMB), jnp.float32),
            pltpu.VMEM((_CH, _EMB), jnp.float32),
            pltpu.VMEM((_CH, _EMB), jnp.float32),
            pltpu.VMEM((_BPW, _EMB), jnp.float32),
            pltpu.SemaphoreType.DMA,
            pltpu.SemaphoreType.DMA,
            pltpu.SemaphoreType.DMA,
            pltpu.SemaphoreType.DMA,
        ],
        compiler_params=pltpu.CompilerParams(use_tc_tiling_on_sc=False),
        name=f"pool_half{half}",
    )
    def _pool(x_hbm, table_hbm, out_hbm, idx_v, rows0_v, rows1_v, rows2_v,
              rows3_v, pooled_v, sem0, sem1, sem2, sem3):
        wid = lax.axis_index("s") * _NC + lax.axis_index("c")
        pltpu.sync_copy(x_hbm.at[pl.ds(base_chunk + wid * _CPW, _CPW)], idx_v)
        inv_l = jnp.float32(1.0 / _L)
        bufs = (rows0_v, rows1_v, rows2_v, rows3_v)
        sems = (sem0, sem1, sem2, sem3)

        def _fire(j, buf, sem):
            pltpu.async_copy(table_hbm.at[idx_v.at[j]], buf, sem)

        def _wait(j, buf, sem):
            pltpu.make_async_copy(table_hbm.at[idx_v.at[j]], buf, sem).wait()

        def _accum(buf):
            def body(l, c):
                b = l * 4
                new = []
                for u in range(4):
                    new.append(c[2 * u] + buf[b + u, pl.ds(0, _LANES)])
                    new.append(c[2 * u + 1] +
                               buf[b + u, pl.ds(_LANES, _LANES)])
                return tuple(new)
            z = jnp.zeros((_LANES,), jnp.float32)
            c = lax.fori_loop(0, _CH // 4, body, (z,) * 8)
            return ((c[0] + c[2]) + (c[4] + c[6]),
                    (c[1] + c[3]) + (c[5] + c[7]))

        _fire(0, bufs[0], sems[0])
        _fire(1, bufs[1], sems[1])
        _fire(2, bufs[2], sems[2])
        _fire(3, bufs[3], sems[3])

        def pair_body(k, carry):
            j = 4 * k
            for h in range(2):
                jc = j + 2 * h
                b0, b1 = bufs[2 * h], bufs[2 * h + 1]
                s0, s1 = sems[2 * h], sems[2 * h + 1]
                _wait(jc, b0, s0)
                a_lo, a_hi = _accum(b0)
                _wait(jc + 1, b1, s1)
                c_lo, c_hi = _accum(b1)

                @pl.when(k < _BPW // 2 - 1)
                def _prefetch():
                    _fire(jc + 4, b0, s0)
                    _fire(jc + 5, b1, s1)

                pooled_v[2 * k + h, pl.ds(0, _LANES)] = (a_lo + c_lo) * inv_l
                pooled_v[2 * k + h, pl.ds(_LANES, _LANES)] = \
                    (a_hi + c_hi) * inv_l
            return carry

        lax.fori_loop(0, _BPW // 2, pair_body, 0)
        pltpu.sync_copy(pooled_v, out_hbm.at[pl.ds(wid * _BPW, _BPW)])

    return _pool


_pools = [_make_pool(h) for h in range(_NHALF)]

_VT = 4096


def _fc_body(p_ref, w_ref, b_ref, o_ref):
    o_ref[...] = lax.dot_general(
        p_ref[...], w_ref[...],
        dimension_numbers=(((1,), (1,)), ((), ())),
        preferred_element_type=jnp.float32,
    ) + b_ref[...]


def _fc_alias_body(p_ref, w_ref, b_ref, _prev_ref, o_ref):
    _fc_body(p_ref, w_ref, b_ref, o_ref)


def _fc_half(half, pooled, fc_w, fc_b2, prev=None):
    in_specs = [
        pl.BlockSpec((_BH, _EMB), lambda i: (0, 0)),
        pl.BlockSpec((_VT, _EMB), lambda i: (i, 0)),
        pl.BlockSpec((1, _VT), lambda i: (0, i)),
    ]
    args = [pooled, fc_w, fc_b2]
    body = _fc_body
    aliases = {}
    if prev is not None:
        in_specs.append(pl.BlockSpec(memory_space=pl.ANY))
        args.append(prev)
        body = _fc_alias_body
        aliases = {3: 0}
    return pl.pallas_call(
        body,
        grid=(pl.cdiv(_VOCAB, _VT),),
        in_specs=in_specs,
        out_specs=pl.BlockSpec((_BH, _VT), lambda i, _h=half: (_h, i)),
        out_shape=jax.ShapeDtypeStruct((_B, _VOCAB), jnp.float32),
        input_output_aliases=aliases,
    )(*args)


def kernel(x, emb_table, fc_w, fc_b):
    xi = x.astype(jnp.int32).reshape(_B * _L // _CH, _CH)
    fc_b2 = fc_b.reshape(1, _VOCAB)
    pooled0 = _pools[0](xi, emb_table)
    pooled1 = _pools[1](xi, emb_table)
    y = _fc_half(0, pooled0, fc_w, fc_b2)
    y = _fc_half(1, pooled1, fc_w, fc_b2, prev=y)
    return y


# speedup vs baseline: 1.0353x; 1.0353x over previous
import functools

import jax
import jax.numpy as jnp
from jax import lax
from jax.experimental import pallas as pl
from jax.experimental.pallas import tpu as pltpu
from jax.experimental.pallas import tpu_sc as plsc

_VOCAB = 100000
_EMB = 32
_B = 1024
_L = 200

_NC = 2
_NS = 16
_NW = _NC * _NS
_CH = 100
_LANES = 16
_NHALF = 1
_BH = _B // _NHALF
_BPW = _BH // _NW
_CPW = _BPW * _L // _CH

_mesh = plsc.VectorSubcoreMesh(core_axis_name="c", subcore_axis_name="s")


def _make_pool(half):
    base_chunk = half * (_BH * _L // _CH)

    @functools.partial(
        pl.kernel,
        mesh=_mesh,
        out_type=jax.ShapeDtypeStruct((_BH, _EMB), jnp.float32),
        scratch_types=(
            [pltpu.VMEM((_CPW, _CH), jnp.int32)]
            + [pltpu.VMEM((_CH, _EMB), jnp.float32) for _ in range(8)]
            + [pltpu.VMEM((_BPW, _EMB), jnp.float32)]
            + [pltpu.SemaphoreType.DMA for _ in range(8)]
        ),
        compiler_params=pltpu.CompilerParams(use_tc_tiling_on_sc=False),
        name=f"pool_half{half}",
    )
    def _pool(x_hbm, table_hbm, out_hbm, idx_v, *rest):
        bufs = rest[:8]
        pooled_v = rest[8]
        sems = rest[9:17]
        wid = lax.axis_index("s") * _NC + lax.axis_index("c")
        pltpu.sync_copy(x_hbm.at[pl.ds(base_chunk + wid * _CPW, _CPW)], idx_v)
        inv_l = jnp.float32(1.0 / _L)

        def _fire(j, buf, sem):
            pltpu.async_copy(table_hbm.at[idx_v.at[j]], buf, sem)

        def _wait(j, buf, sem):
            pltpu.make_async_copy(table_hbm.at[idx_v.at[j]], buf, sem).wait()

        def _accum(buf):
            def body(l, c):
                b = l * 4
                new = []
                for u in range(4):
                    new.append(c[2 * u] + buf[b + u, pl.ds(0, _LANES)])
                    new.append(c[2 * u + 1] +
                               buf[b + u, pl.ds(_LANES, _LANES)])
                return tuple(new)
            z = jnp.zeros((_LANES,), jnp.float32)
            c = lax.fori_loop(0, _CH // 4, body, (z,) * 8)
            return ((c[0] + c[2]) + (c[4] + c[6]),
                    (c[1] + c[3]) + (c[5] + c[7]))

        for j in range(8):
            _fire(j, bufs[j], sems[j])

        def group_body(k, carry):
            j = 8 * k
            for q in range(4):
                jc = j + 2 * q
                b0, b1 = bufs[2 * q], bufs[2 * q + 1]
                s0, s1 = sems[2 * q], sems[2 * q + 1]
                _wait(jc, b0, s0)
                a_lo, a_hi = _accum(b0)
                _wait(jc + 1, b1, s1)
                c_lo, c_hi = _accum(b1)

                @pl.when(k < _BPW // 4 - 1)
                def _prefetch():
                    _fire(jc + 8, b0, s0)
                    _fire(jc + 9, b1, s1)

                pooled_v[4 * k + q, pl.ds(0, _LANES)] = (a_lo + c_lo) * inv_l
                pooled_v[4 * k + q, pl.ds(_LANES, _LANES)] = \
                    (a_hi + c_hi) * inv_l
            return carry

        lax.fori_loop(0, _BPW // 4, group_body, 0)
        pltpu.sync_copy(pooled_v, out_hbm.at[pl.ds(wid * _BPW, _BPW)])

    return _pool


_pools = [_make_pool(h) for h in range(_NHALF)]

_VT = 4096


def _fc_body(p_ref, w_ref, b_ref, o_ref):
    o_ref[...] = lax.dot_general(
        p_ref[...], w_ref[...],
        dimension_numbers=(((1,), (1,)), ((), ())),
        preferred_element_type=jnp.float32,
    ) + b_ref[...]


def _fc_alias_body(p_ref, w_ref, b_ref, _prev_ref, o_ref):
    _fc_body(p_ref, w_ref, b_ref, o_ref)


def _fc_half(half, pooled, fc_w, fc_b2, prev=None):
    in_specs = [
        pl.BlockSpec((_BH, _EMB), lambda i: (0, 0)),
        pl.BlockSpec((_VT, _EMB), lambda i: (i, 0)),
        pl.BlockSpec((1, _VT), lambda i: (0, i)),
    ]
    args = [pooled, fc_w, fc_b2]
    body = _fc_body
    aliases = {}
    if prev is not None:
        in_specs.append(pl.BlockSpec(memory_space=pl.ANY))
        args.append(prev)
        body = _fc_alias_body
        aliases = {3: 0}
    return pl.pallas_call(
        body,
        grid=(pl.cdiv(_VOCAB, _VT),),
        in_specs=in_specs,
        out_specs=pl.BlockSpec((_BH, _VT), lambda i, _h=half: (_h, i)),
        out_shape=jax.ShapeDtypeStruct((_B, _VOCAB), jnp.float32),
        input_output_aliases=aliases,
    )(*args)


def kernel(x, emb_table, fc_w, fc_b):
    xi = x.astype(jnp.int32).reshape(_B * _L // _CH, _CH)
    fc_b2 = fc_b.reshape(1, _VOCAB)
    pooled = _pools[0](xi, emb_table)
    return _fc_half(0, pooled, fc_w, fc_b2)


# speedup vs baseline: 1.0403x; 1.0048x over previous
import functools

import jax
import jax.numpy as jnp
from jax import lax
from jax.experimental import pallas as pl
from jax.experimental.pallas import tpu as pltpu
from jax.experimental.pallas import tpu_sc as plsc

_VOCAB = 100000
_EMB = 32
_B = 1024
_L = 200

_NC = 2
_NS = 16
_NW = _NC * _NS
_CH = 100
_LANES = 16
_NHALF = 1
_BH = _B // _NHALF
_BPW = _BH // _NW
_CPW = _BPW * _L // _CH

_mesh = plsc.VectorSubcoreMesh(core_axis_name="c", subcore_axis_name="s")


def _make_pool(half):
    base_chunk = half * (_BH * _L // _CH)

    @functools.partial(
        pl.kernel,
        mesh=_mesh,
        out_type=jax.ShapeDtypeStruct((_BH, _EMB), jnp.float32),
        scratch_types=(
            [pltpu.VMEM((_CPW, _CH), jnp.int32)]
            + [pltpu.VMEM((_CH, _EMB), jnp.float32) for _ in range(8)]
            + [pltpu.VMEM((_BPW, _EMB), jnp.float32)]
            + [pltpu.SemaphoreType.DMA for _ in range(8)]
        ),
        compiler_params=pltpu.CompilerParams(use_tc_tiling_on_sc=False),
        name=f"pool_half{half}",
    )
    def _pool(x_hbm, table_hbm, out_hbm, idx_v, *rest):
        bufs = rest[:8]
        pooled_v = rest[8]
        sems = rest[9:17]
        wid = lax.axis_index("s") * _NC + lax.axis_index("c")
        pltpu.sync_copy(x_hbm.at[pl.ds(base_chunk + wid * _CPW, _CPW)], idx_v)
        inv_l = jnp.float32(1.0 / _L)

        def _fire(j, buf, sem):
            pltpu.async_copy(table_hbm.at[idx_v.at[j]], buf, sem)

        def _wait(j, buf, sem):
            pltpu.make_async_copy(table_hbm.at[idx_v.at[j]], buf, sem).wait()

        def _accum(buf):
            def body(l, c):
                b = l * 4
                new = []
                for u in range(4):
                    new.append(c[2 * u] + buf[b + u, pl.ds(0, _LANES)])
                    new.append(c[2 * u + 1] +
                               buf[b + u, pl.ds(_LANES, _LANES)])
                return tuple(new)
            z = jnp.zeros((_LANES,), jnp.float32)
            c = lax.fori_loop(0, _CH // 4, body, (z,) * 8)
            return ((c[0] + c[2]) + (c[4] + c[6]),
                    (c[1] + c[3]) + (c[5] + c[7]))

        for j in range(4):
            _fire(j, bufs[j], sems[j])

        def pair_body(k, carry):
            j = 4 * k
            for h in range(2):
                jc = j + 2 * h
                b0, b1 = bufs[2 * h], bufs[2 * h + 1]
                s0, s1 = sems[2 * h], sems[2 * h + 1]
                _wait(jc, b0, s0)
                a_lo, a_hi = _accum(b0)
                _wait(jc + 1, b1, s1)
                c_lo, c_hi = _accum(b1)

                @pl.when(k < _BPW // 2 - 1)
                def _prefetch():
                    _fire(jc + 4, b0, s0)
                    _fire(jc + 5, b1, s1)

                pooled_v[2 * k + h, pl.ds(0, _LANES)] = (a_lo + c_lo) * inv_l
                pooled_v[2 * k + h, pl.ds(_LANES, _LANES)] = \
                    (a_hi + c_hi) * inv_l
            return carry

        lax.fori_loop(0, _BPW // 2, pair_body, 0)
        pltpu.sync_copy(pooled_v, out_hbm.at[pl.ds(wid * _BPW, _BPW)])

    return _pool


_pools = [_make_pool(h) for h in range(_NHALF)]

_VT = 4096


def _fc_body(p_ref, w_ref, b_ref, o_ref):
    o_ref[...] = lax.dot_general(
        p_ref[...], w_ref[...],
        dimension_numbers=(((1,), (1,)), ((), ())),
        preferred_element_type=jnp.float32,
    ) + b_ref[...]


def _fc_alias_body(p_ref, w_ref, b_ref, _prev_ref, o_ref):
    _fc_body(p_ref, w_ref, b_ref, o_ref)


def _fc_half(half, pooled, fc_w, fc_b2, prev=None):
    in_specs = [
        pl.BlockSpec((_BH, _EMB), lambda i: (0, 0)),
        pl.BlockSpec((_VT, _EMB), lambda i: (i, 0)),
        pl.BlockSpec((1, _VT), lambda i: (0, i)),
    ]
    args = [pooled, fc_w, fc_b2]
    body = _fc_body
    aliases = {}
    if prev is not None:
        in_specs.append(pl.BlockSpec(memory_space=pl.ANY))
        args.append(prev)
        body = _fc_alias_body
        aliases = {3: 0}
    return pl.pallas_call(
        body,
        grid=(pl.cdiv(_VOCAB, _VT),),
        in_specs=in_specs,
        out_specs=pl.BlockSpec((_BH, _VT), lambda i, _h=half: (_h, i)),
        out_shape=jax.ShapeDtypeStruct((_B, _VOCAB), jnp.float32),
        input_output_aliases=aliases,
    )(*args)


def kernel(x, emb_table, fc_w, fc_b):
    xi = x.astype(jnp.int32).reshape(_B * _L // _CH, _CH)
    fc_b2 = fc_b.reshape(1, _VOCAB)
    pooled = _pools[0](xi, emb_table)
    return _fc_half(0, pooled, fc_w, fc_b2)


# speedup vs baseline: 1.0426x; 1.0023x over previous
import functools

import jax
import jax.numpy as jnp
from jax import lax
from jax.experimental import pallas as pl
from jax.experimental.pallas import tpu as pltpu
from jax.experimental.pallas import tpu_sc as plsc

_VOCAB = 100000
_EMB = 32
_B = 1024
_L = 200

_NC = 2
_NS = 16
_NW = _NC * _NS
_CH = 100
_LANES = 16
_NHALF = 1
_BH = _B // _NHALF
_BPW = _BH // _NW
_CPW = _BPW * _L // _CH

_mesh = plsc.VectorSubcoreMesh(core_axis_name="c", subcore_axis_name="s")


def _make_pool(half):
    base_chunk = half * (_BH * _L // _CH)

    @functools.partial(
        pl.kernel,
        mesh=_mesh,
        out_type=jax.ShapeDtypeStruct((_BH, _EMB), jnp.float32),
        scratch_types=(
            [pltpu.VMEM((_CPW, _CH), jnp.int32)]
            + [pltpu.VMEM((_L, _EMB), jnp.float32) for _ in range(2)]
            + [pltpu.VMEM((_BPW, _EMB), jnp.float32)]
            + [pltpu.SemaphoreType.DMA for _ in range(2)]
        ),
        compiler_params=pltpu.CompilerParams(use_tc_tiling_on_sc=False),
        name=f"pool_half{half}",
    )
    def _pool(x_hbm, table_hbm, out_hbm, idx_v, buf_a, buf_b, pooled_v,
              sem_a, sem_b):
        bufs = (buf_a, buf_b)
        sems = (sem_a, sem_b)
        wid = lax.axis_index("s") * _NC + lax.axis_index("c")
        pltpu.sync_copy(x_hbm.at[pl.ds(base_chunk + wid * _CPW, _CPW)], idx_v)
        inv_l = jnp.float32(1.0 / _L)

        def _fire_row(r, buf, sem):
            pltpu.async_copy(table_hbm.at[idx_v.at[2 * r]],
                             buf.at[pl.ds(0, _CH)], sem)
            pltpu.async_copy(table_hbm.at[idx_v.at[2 * r + 1]],
                             buf.at[pl.ds(_CH, _CH)], sem)

        def _wait_row(buf, sem):
            pltpu.make_async_copy(table_hbm.at[pl.ds(0, _L)], buf, sem).wait()

        def _accum(buf):
            def body(l, c):
                b = l * 8
                new = list(c)
                for u in range(8):
                    new[(2 * u) % 8] = (new[(2 * u) % 8] +
                                        buf[b + u, pl.ds(0, _LANES)])
                    new[(2 * u + 1) % 8] = (new[(2 * u + 1) % 8] +
                                            buf[b + u, pl.ds(_LANES, _LANES)])
                return tuple(new)
            z = jnp.zeros((_LANES,), jnp.float32)
            c = lax.fori_loop(0, _L // 8, body, (z,) * 8)
            return ((c[0] + c[2]) + (c[4] + c[6]),
                    (c[1] + c[3]) + (c[5] + c[7]))

        _fire_row(0, bufs[0], sems[0])
        _fire_row(1, bufs[1], sems[1])

        def pair_body(k, carry):
            for h in range(2):
                r = 2 * k + h
                buf, sem = bufs[h], sems[h]
                _wait_row(buf, sem)
                a_lo, a_hi = _accum(buf)

                @pl.when(k < _BPW // 2 - 1)
                def _prefetch():
                    _fire_row(r + 2, buf, sem)

                pooled_v[r, pl.ds(0, _LANES)] = a_lo * inv_l
                pooled_v[r, pl.ds(_LANES, _LANES)] = a_hi * inv_l
            return carry

        lax.fori_loop(0, _BPW // 2, pair_body, 0)
        pltpu.sync_copy(pooled_v, out_hbm.at[pl.ds(wid * _BPW, _BPW)])

    return _pool


_pools = [_make_pool(h) for h in range(_NHALF)]

_VT = 4096


def _fc_body(p_ref, w_ref, b_ref, o_ref):
    o_ref[...] = lax.dot_general(
        p_ref[...], w_ref[...],
        dimension_numbers=(((1,), (1,)), ((), ())),
        preferred_element_type=jnp.float32,
    ) + b_ref[...]


def _fc_alias_body(p_ref, w_ref, b_ref, _prev_ref, o_ref):
    _fc_body(p_ref, w_ref, b_ref, o_ref)


def _fc_half(half, pooled, fc_w, fc_b2, prev=None):
    in_specs = [
        pl.BlockSpec((_BH, _EMB), lambda i: (0, 0)),
        pl.BlockSpec((_VT, _EMB), lambda i: (i, 0)),
        pl.BlockSpec((1, _VT), lambda i: (0, i)),
    ]
    args = [pooled, fc_w, fc_b2]
    body = _fc_body
    aliases = {}
    if prev is not None:
        in_specs.append(pl.BlockSpec(memory_space=pl.ANY))
        args.append(prev)
        body = _fc_alias_body
        aliases = {3: 0}
    return pl.pallas_call(
        body,
        grid=(pl.cdiv(_VOCAB, _VT),),
        in_specs=in_specs,
        out_specs=pl.BlockSpec((_BH, _VT), lambda i, _h=half: (_h, i)),
        out_shape=jax.ShapeDtypeStruct((_B, _VOCAB), jnp.float32),
        input_output_aliases=aliases,
    )(*args)


def kernel(x, emb_table, fc_w, fc_b):
    xi = x.astype(jnp.int32).reshape(_B * _L // _CH, _CH)
    fc_b2 = fc_b.reshape(1, _VOCAB)
    pooled = _pools[0](xi, emb_table)
    return _fc_half(0, pooled, fc_w, fc_b2)


# speedup vs baseline: 1.0785x; 1.0344x over previous
import functools

import jax
import jax.numpy as jnp
from jax import lax
from jax.experimental import pallas as pl
from jax.experimental.pallas import tpu as pltpu
from jax.experimental.pallas import tpu_sc as plsc

_VOCAB = 100000
_EMB = 32
_B = 1024
_L = 200

_NC = 2
_NS = 16
_NW = _NC * _NS
_CH = 100
_LANES = 16
_NHALF = 1
_BH = _B // _NHALF
_BPW = _BH // _NW
_CPW = _BPW * _L // _CH

_mesh = plsc.VectorSubcoreMesh(core_axis_name="c", subcore_axis_name="s")


def _make_pool(half):
    base_chunk = half * (_BH * _L // _CH)

    @functools.partial(
        pl.kernel,
        mesh=_mesh,
        out_type=jax.ShapeDtypeStruct((_BH, _EMB), jnp.float32),
        scratch_types=(
            [pltpu.VMEM((_CPW, _CH), jnp.int32)]
            + [pltpu.VMEM((_L, _EMB), jnp.float32) for _ in range(2)]
            + [pltpu.VMEM((_BPW, _EMB), jnp.float32)]
            + [pltpu.SemaphoreType.DMA for _ in range(2)]
        ),
        compiler_params=pltpu.CompilerParams(use_tc_tiling_on_sc=False),
        name=f"pool_half{half}",
    )
    def _pool(x_hbm, table_hbm, out_hbm, idx_v, buf_a, buf_b, pooled_v,
              sem_a, sem_b):
        bufs = (buf_a, buf_b)
        sems = (sem_a, sem_b)
        wid = lax.axis_index("s") * _NC + lax.axis_index("c")
        pltpu.sync_copy(x_hbm.at[pl.ds(base_chunk + wid * _CPW, _CPW)], idx_v)
        inv_l = jnp.float32(1.0 / _L)

        def _fire_row(r, buf, sem):
            pltpu.async_copy(table_hbm.at[idx_v.at[2 * r]],
                             buf.at[pl.ds(0, _CH)], sem)
            pltpu.async_copy(table_hbm.at[idx_v.at[2 * r + 1]],
                             buf.at[pl.ds(_CH, _CH)], sem)

        def _wait_row(buf, sem):
            pltpu.make_async_copy(table_hbm.at[pl.ds(0, _L)], buf, sem).wait()

        def _accum(buf):
            def body(l, c):
                b = l * 8
                new = list(c)
                for u in range(8):
                    new[(2 * u) % 8] = (new[(2 * u) % 8] +
                                        buf[b + u, pl.ds(0, _LANES)])
                    new[(2 * u + 1) % 8] = (new[(2 * u + 1) % 8] +
                                            buf[b + u, pl.ds(_LANES, _LANES)])
                return tuple(new)
            z = jnp.zeros((_LANES,), jnp.float32)
            c = lax.fori_loop(0, _L // 8, body, (z,) * 8)
            return ((c[0] + c[2]) + (c[4] + c[6]),
                    (c[1] + c[3]) + (c[5] + c[7]))

        _fire_row(0, bufs[0], sems[0])
        _fire_row(1, bufs[1], sems[1])

        def pair_body(k, carry):
            for h in range(2):
                r = 2 * k + h
                buf, sem = bufs[h], sems[h]
                _wait_row(buf, sem)
                a_lo, a_hi = _accum(buf)

                @pl.when(k < _BPW // 2 - 1)
                def _prefetch():
                    _fire_row(r + 2, buf, sem)

                pooled_v[r, pl.ds(0, _LANES)] = a_lo * inv_l
                pooled_v[r, pl.ds(_LANES, _LANES)] = a_hi * inv_l
            return carry

        lax.fori_loop(0, _BPW // 2, pair_body, 0)
        pltpu.sync_copy(pooled_v, out_hbm.at[pl.ds(wid * _BPW, _BPW)])

    return _pool


_pools = [_make_pool(h) for h in range(_NHALF)]

_RB = 32
_NGRID = _B // _RB
_NBUF = 2


def _fc_body(p_ref, w_hbm, b_ref, o_hbm, w_v, ob0, ob1, semw, sem0, sem1):
    i = pl.program_id(0)
    obufs = (ob0, ob1)
    sems = (sem0, sem1)

    @pl.when(i == 0)
    def _load_w():
        cp = pltpu.make_async_copy(w_hbm, w_v, semw)
        cp.start()
        cp.wait()

    for s in range(_NBUF):
        @pl.when(i % _NBUF == s)
        def _slot():
            obuf, sem = obufs[s], sems[s]

            @pl.when(i >= _NBUF)
            def _drain_prev():
                pltpu.make_async_copy(
                    obuf, o_hbm.at[pl.ds(0, _RB)], sem).wait()

            obuf[...] = lax.dot_general(
                p_ref[...], w_v[...],
                dimension_numbers=(((1,), (0,)), ((), ())),
                preferred_element_type=jnp.float32,
            ) + b_ref[...]
            pltpu.make_async_copy(
                obuf, o_hbm.at[pl.ds(i * _RB, _RB)], sem).start()

            @pl.when(i == _NGRID - 1)
            def _final_drain():
                pltpu.make_async_copy(
                    obuf, o_hbm.at[pl.ds(0, _RB)], sem).wait()
                other, osem = obufs[1 - s], sems[1 - s]
                pltpu.make_async_copy(
                    other, o_hbm.at[pl.ds(0, _RB)], osem).wait()


def _fc(pooled, fc_w, fc_b2):
    return pl.pallas_call(
        _fc_body,
        grid=(_NGRID,),
        in_specs=[
            pl.BlockSpec((_RB, _EMB), lambda i: (i, 0)),
            pl.BlockSpec(memory_space=pl.ANY),
            pl.BlockSpec((1, _VOCAB), lambda i: (0, 0)),
        ],
        out_specs=pl.BlockSpec(memory_space=pl.ANY),
        out_shape=jax.ShapeDtypeStruct((_B, _VOCAB), jnp.float32),
        scratch_shapes=[
            pltpu.VMEM((_EMB, _VOCAB), jnp.float32),
            pltpu.VMEM((_RB, _VOCAB), jnp.float32),
            pltpu.VMEM((_RB, _VOCAB), jnp.float32),
            pltpu.SemaphoreType.DMA,
            pltpu.SemaphoreType.DMA,
            pltpu.SemaphoreType.DMA,
        ],
    )(pooled, fc_w, fc_b2)


def kernel(x, emb_table, fc_w, fc_b):
    xi = x.astype(jnp.int32).reshape(_B * _L // _CH, _CH)
    fc_b2 = fc_b.reshape(1, _VOCAB)
    pooled = _pools[0](xi, emb_table)
    return _fc(pooled, jnp.swapaxes(fc_w, 0, 1), fc_b2)


# speedup vs baseline: 1.0817x; 1.0030x over previous
import functools

import jax
import jax.numpy as jnp
from jax import lax
from jax.experimental import pallas as pl
from jax.experimental.pallas import tpu as pltpu
from jax.experimental.pallas import tpu_sc as plsc

_VOCAB = 100000
_EMB = 32
_B = 1024
_L = 200

_NC = 2
_NS = 16
_NW = _NC * _NS
_CH = 100
_LANES = 16
_NHALF = 1
_BH = _B // _NHALF
_BPW = _BH // _NW
_CPW = _BPW * _L // _CH

_mesh = plsc.VectorSubcoreMesh(core_axis_name="c", subcore_axis_name="s")


def _make_pool(half):
    base_chunk = half * (_BH * _L // _CH)

    @functools.partial(
        pl.kernel,
        mesh=_mesh,
        out_type=jax.ShapeDtypeStruct((_BH, _EMB), jnp.float32),
        scratch_types=(
            [pltpu.VMEM((_CPW, _CH), jnp.int32)]
            + [pltpu.VMEM((_L, _EMB), jnp.float32) for _ in range(2)]
            + [pltpu.VMEM((_BPW, _EMB), jnp.float32)]
            + [pltpu.SemaphoreType.DMA for _ in range(2)]
        ),
        compiler_params=pltpu.CompilerParams(use_tc_tiling_on_sc=False),
        name=f"pool_half{half}",
    )
    def _pool(x_hbm, table_hbm, out_hbm, idx_v, buf_a, buf_b, pooled_v,
              sem_a, sem_b):
        bufs = (buf_a, buf_b)
        sems = (sem_a, sem_b)
        wid = lax.axis_index("s") * _NC + lax.axis_index("c")
        pltpu.sync_copy(x_hbm.at[pl.ds(base_chunk + wid * _CPW, _CPW)], idx_v)
        inv_l = jnp.float32(1.0 / _L)

        def _fire_row(r, buf, sem):
            pltpu.async_copy(table_hbm.at[idx_v.at[2 * r]],
                             buf.at[pl.ds(0, _CH)], sem)
            pltpu.async_copy(table_hbm.at[idx_v.at[2 * r + 1]],
                             buf.at[pl.ds(_CH, _CH)], sem)

        def _wait_row(buf, sem):
            pltpu.make_async_copy(table_hbm.at[pl.ds(0, _L)], buf, sem).wait()

        def _accum(buf):
            def body(l, c):
                b = l * 8
                new = list(c)
                for u in range(8):
                    new[(2 * u) % 8] = (new[(2 * u) % 8] +
                                        buf[b + u, pl.ds(0, _LANES)])
                    new[(2 * u + 1) % 8] = (new[(2 * u + 1) % 8] +
                                            buf[b + u, pl.ds(_LANES, _LANES)])
                return tuple(new)
            z = jnp.zeros((_LANES,), jnp.float32)
            c = lax.fori_loop(0, _L // 8, body, (z,) * 8)
            return ((c[0] + c[2]) + (c[4] + c[6]),
                    (c[1] + c[3]) + (c[5] + c[7]))

        _fire_row(0, bufs[0], sems[0])
        _fire_row(1, bufs[1], sems[1])

        def pair_body(k, carry):
            for h in range(2):
                r = 2 * k + h
                buf, sem = bufs[h], sems[h]
                _wait_row(buf, sem)
                a_lo, a_hi = _accum(buf)

                @pl.when(k < _BPW // 2 - 1)
                def _prefetch():
                    _fire_row(r + 2, buf, sem)

                pooled_v[r, pl.ds(0, _LANES)] = a_lo * inv_l
                pooled_v[r, pl.ds(_LANES, _LANES)] = a_hi * inv_l
            return carry

        lax.fori_loop(0, _BPW // 2, pair_body, 0)
        pltpu.sync_copy(pooled_v, out_hbm.at[pl.ds(wid * _BPW, _BPW)])

    return _pool


_pools = [_make_pool(h) for h in range(_NHALF)]

_RB = 32
_NGRID = _B // _RB
_NBUF = 3


def _fc_body(p_ref, w_hbm, b_ref, o_hbm, w_v, ob0, ob1, ob2, semw,
             sem0, sem1, sem2):
    i = pl.program_id(0)
    obufs = (ob0, ob1, ob2)
    sems = (sem0, sem1, sem2)

    @pl.when(i == 0)
    def _load_w():
        cp = pltpu.make_async_copy(w_hbm, w_v, semw)
        cp.start()
        cp.wait()

    for s in range(_NBUF):
        @pl.when(i % _NBUF == s)
        def _slot():
            obuf, sem = obufs[s], sems[s]

            @pl.when(i >= _NBUF)
            def _drain_prev():
                pltpu.make_async_copy(
                    obuf, o_hbm.at[pl.ds(0, _RB)], sem).wait()

            obuf[...] = lax.dot_general(
                p_ref[...], w_v[...],
                dimension_numbers=(((1,), (0,)), ((), ())),
                preferred_element_type=jnp.float32,
            ) + b_ref[...]
            pltpu.make_async_copy(
                obuf, o_hbm.at[pl.ds(i * _RB, _RB)], sem).start()

            @pl.when(i == _NGRID - 1)
            def _final_drain():
                for t in range(_NBUF):
                    pltpu.make_async_copy(
                        obufs[(s + t) % _NBUF], o_hbm.at[pl.ds(0, _RB)],
                        sems[(s + t) % _NBUF]).wait()


def _fc(pooled, fc_w, fc_b2):
    return pl.pallas_call(
        _fc_body,
        grid=(_NGRID,),
        in_specs=[
            pl.BlockSpec((_RB, _EMB), lambda i: (i, 0)),
            pl.BlockSpec(memory_space=pl.ANY),
            pl.BlockSpec((1, _VOCAB), lambda i: (0, 0)),
        ],
        out_specs=pl.BlockSpec(memory_space=pl.ANY),
        out_shape=jax.ShapeDtypeStruct((_B, _VOCAB), jnp.float32),
        scratch_shapes=(
            [pltpu.VMEM((_EMB, _VOCAB), jnp.float32)]
            + [pltpu.VMEM((_RB, _VOCAB), jnp.float32)
               for _ in range(_NBUF)]
            + [pltpu.SemaphoreType.DMA for _ in range(_NBUF + 1)]
        ),
    )(pooled, fc_w, fc_b2)


def kernel(x, emb_table, fc_w, fc_b):
    xi = x.astype(jnp.int32).reshape(_B * _L // _CH, _CH)
    fc_b2 = fc_b.reshape(1, _VOCAB)
    pooled = _pools[0](xi, emb_table)
    return _fc(pooled, jnp.swapaxes(fc_w, 0, 1), fc_b2)
